# baseline (speedup 1.0000x reference)
import functools

import jax
import jax.numpy as jnp
from jax import lax
from jax.experimental import pallas as pl
from jax.experimental.pallas import tpu as pltpu
from jax.experimental.pallas import tpu_sc as plsc

N = 10000
F = 128
D = 128
KHOP = 3
E = 320000
NC, NS, L = 2, 16, 16
NW = NC * NS
ET = KHOP * E
CROWS = 2
CHUNK = 192
ETP = ((ET + NW * CHUNK - 1) // (NW * CHUNK)) * NW * CHUNK
EPW = ETP // NW
NCHUNK = EPW // CHUNK
NP = 10240
ROWS_PER_TILE = NP // NS

BN = 1000


def _dense_body(x_ref, w1_ref, b1_ref, w2_ref, b2_ref, wb_ref, wf_ref,
                wr_ref, a_ref, mn_ref, dense_ref):
    x = x_ref[...]
    h = jnp.maximum(
        jnp.dot(x, w1_ref[...], preferred_element_type=jnp.float32)
        + b1_ref[...][None, :], 0.0)
    le = (jnp.dot(h, w2_ref[...], preferred_element_type=jnp.float32)
          + b2_ref[...][None, :])
    a = jnp.clip(a_ref[...], 0.0, 1.0)
    wcomb = jnp.sum(a) * wb_ref[...] + jnp.sum(
        a[:, None, None] * wr_ref[...], axis=0)
    dense_ref[...] = jnp.dot(le, wcomb, preferred_element_type=jnp.float32)
    for hop in range(KHOP):
        mn_ref[hop] = jnp.dot(x, wf_ref[hop],
                              preferred_element_type=jnp.float32)


def _dense_call(x, w1, b1, w2, b2, wb, wf, wr, a):
    grid = (N // BN,)
    full = lambda shape: pl.BlockSpec(shape, lambda i: tuple(0 for _ in shape))
    return pl.pallas_call(
        _dense_body,
        grid=grid,
        in_specs=[
            pl.BlockSpec((BN, F), lambda i: (i, 0)),
            full((F, 2 * D)),
            full((2 * D,)),
            full((2 * D, D)),
            full((D,)),
            full((D, D)),
            full((KHOP, F, D)),
            full((KHOP, D, D)),
            full((KHOP,)),
        ],
        out_specs=[
            pl.BlockSpec((KHOP, BN, D), lambda i: (0, i, 0)),
            pl.BlockSpec((BN, D), lambda i: (i, 0)),
        ],
        out_shape=[
            jax.ShapeDtypeStruct((KHOP, N, D), jnp.float32),
            jax.ShapeDtypeStruct((N, D), jnp.float32),
        ],
    )(x, w1, b1, w2, b2, wb, wf, wr, a)


def _sc_body(mn_hbm, srcg_hbm, dst_hbm, val_hbm, zeros_hbm, out_hbm,
             acc_sh, idx_v, dst_v, val_v, rows_v, sem):
    c = lax.axis_index("c")
    s = lax.axis_index("s")
    w = s * NC + c

    pltpu.sync_copy(zeros_hbm.at[pl.ds(s * ROWS_PER_TILE, ROWS_PER_TILE)],
                    acc_sh.at[pl.ds(s * ROWS_PER_TILE, ROWS_PER_TILE)])
    plsc.subcore_barrier()

    def chunk_body(i, carry):
        base = w * EPW + i * CHUNK
        pltpu.sync_copy(srcg_hbm.at[pl.ds(base, CHUNK)], idx_v)
        pltpu.sync_copy(dst_hbm.at[pl.ds(base, CHUNK)], dst_v)
        pltpu.sync_copy(val_hbm.at[pl.ds(base, CHUNK)], val_v)
        pltpu.async_copy(mn_hbm.at[idx_v], rows_v, sem).wait()

        def scale_group(g, carry2):
            grp = val_v[pl.ds(g * L, L)]
            for lane in range(L):
                v = grp[lane]
                e = g * L + lane
                for j in range(D // L):
                    sl = pl.ds(j * L, L)
                    rows_v[e, sl] = rows_v[e, sl] * v
            return carry2

        lax.fori_loop(0, CHUNK // L, scale_group, 0)
        pltpu.sync_copy(rows_v, acc_sh.at[dst_v], add=True)
        return carry

    lax.fori_loop(0, NCHUNK, chunk_body, 0)
    plsc.subcore_barrier()

    pltpu.sync_copy(
        acc_sh.at[pl.ds(s * ROWS_PER_TILE, ROWS_PER_TILE)],
        out_hbm.at[pl.ds(c * NP + s * ROWS_PER_TILE, ROWS_PER_TILE)])


_sc_call = pl.kernel(
    _sc_body,
    out_type=jax.ShapeDtypeStruct((NC * NP, D), jnp.float32),
    mesh=plsc.VectorSubcoreMesh(core_axis_name="c", subcore_axis_name="s"),
    scratch_types=[
        pltpu.VMEM_SHARED((NP, D), jnp.float32),
        pltpu.VMEM((CHUNK,), jnp.int32),
        pltpu.VMEM((CHUNK,), jnp.int32),
        pltpu.VMEM((CHUNK,), jnp.float32),
        pltpu.VMEM((CHUNK, D), jnp.float32),
        pltpu.SemaphoreType.DMA,
    ],
)


def _finish_body(acc_ref, dense_ref, out_ref):
    out_ref[...] = jnp.maximum(acc_ref[0] + acc_ref[1] + dense_ref[...], 0.0)


def _finish_call(accs, dense):
    return pl.pallas_call(
        _finish_body,
        grid=(N // BN,),
        in_specs=[
            pl.BlockSpec((NC, BN, D), lambda i: (0, i, 0)),
            pl.BlockSpec((BN, D), lambda i: (i, 0)),
        ],
        out_specs=pl.BlockSpec((BN, D), lambda i: (i, 0)),
        out_shape=jax.ShapeDtypeStruct((N, D), jnp.float32),
    )(accs, dense)


def kernel(node_features, edge_index, adj_values, W_emb1, b_emb1, W_emb2,
           b_emb2, W_base, W_feat, W_res, alpha):
    mn, dense = _dense_call(node_features, W_emb1, b_emb1, W_emb2, b_emb2,
                            W_base, W_feat, W_res, alpha)
    mn_flat = mn.reshape(KHOP * N, D)

    src = edge_index[:, 0, :]
    dst = edge_index[:, 1, :]
    srcg = (src + (jnp.arange(KHOP, dtype=jnp.int32) * N)[:, None]).reshape(-1)
    dstf = dst.reshape(-1)
    valf = adj_values.reshape(-1)
    pad = ETP - ET
    srcg = jnp.concatenate([srcg, jnp.zeros((pad,), jnp.int32)])
    dstf = jnp.concatenate([dstf, jnp.zeros((pad,), jnp.int32)])
    valf = jnp.concatenate([valf, jnp.zeros((pad,), jnp.float32)])
    zeros = jnp.zeros((NP, D), jnp.float32)

    accs = _sc_call(mn_flat, srcg, dstf, valf, zeros)
    accs = accs.reshape(NC, NP, D)[:, :N, :]
    return _finish_call(accs, dense)


# speedup vs baseline: 1.2482x; 1.2482x over previous
import functools

import jax
import jax.numpy as jnp
from jax import lax
from jax.experimental import pallas as pl
from jax.experimental.pallas import tpu as pltpu
from jax.experimental.pallas import tpu_sc as plsc

N = 10000
F = 128
D = 128
KHOP = 3
E = 320000
NC, NS, L = 2, 16, 16
NW = NC * NS
ET = KHOP * E
CROWS = 2
CHUNK = 320
ETP = ((ET + NW * CHUNK - 1) // (NW * CHUNK)) * NW * CHUNK
EPW = ETP // NW
NCHUNK = EPW // CHUNK
NP = 10240
ROWS_PER_TILE = NP // NS

BN = 1000


def _dense_body(x_ref, w1_ref, b1_ref, w2_ref, b2_ref, wb_ref, wf_ref,
                wr_ref, a_ref, mn_ref, dense_ref):
    x = x_ref[...]
    h = jnp.maximum(
        jnp.dot(x, w1_ref[...], preferred_element_type=jnp.float32)
        + b1_ref[...][None, :], 0.0)
    le = (jnp.dot(h, w2_ref[...], preferred_element_type=jnp.float32)
          + b2_ref[...][None, :])
    a = jnp.clip(a_ref[...], 0.0, 1.0)
    wcomb = jnp.sum(a) * wb_ref[...] + jnp.sum(
        a[:, None, None] * wr_ref[...], axis=0)
    dense_ref[...] = jnp.dot(le, wcomb, preferred_element_type=jnp.float32)
    for hop in range(KHOP):
        mn_ref[hop] = jnp.dot(x, wf_ref[hop],
                              preferred_element_type=jnp.float32)


def _dense_call(x, w1, b1, w2, b2, wb, wf, wr, a):
    grid = (N // BN,)
    full = lambda shape: pl.BlockSpec(shape, lambda i: tuple(0 for _ in shape))
    return pl.pallas_call(
        _dense_body,
        grid=grid,
        in_specs=[
            pl.BlockSpec((BN, F), lambda i: (i, 0)),
            full((F, 2 * D)),
            full((2 * D,)),
            full((2 * D, D)),
            full((D,)),
            full((D, D)),
            full((KHOP, F, D)),
            full((KHOP, D, D)),
            full((KHOP,)),
        ],
        out_specs=[
            pl.BlockSpec((KHOP, BN, D), lambda i: (0, i, 0)),
            pl.BlockSpec((BN, D), lambda i: (i, 0)),
        ],
        out_shape=[
            jax.ShapeDtypeStruct((KHOP, N, D), jnp.float32),
            jax.ShapeDtypeStruct((N, D), jnp.float32),
        ],
    )(x, w1, b1, w2, b2, wb, wf, wr, a)


def _sc_body(mn_hbm, srcg_hbm, dst_hbm, val_hbm, zeros_hbm, out_hbm,
             acc_sh, idx_v, dst_v, val_v, rows_v, sem):
    c = lax.axis_index("c")
    s = lax.axis_index("s")
    w = s * NC + c

    pltpu.sync_copy(zeros_hbm.at[pl.ds(s * ROWS_PER_TILE, ROWS_PER_TILE)],
                    acc_sh.at[pl.ds(s * ROWS_PER_TILE, ROWS_PER_TILE)])
    plsc.subcore_barrier()

    def chunk_body(i, carry):
        base = w * EPW + i * CHUNK
        pltpu.sync_copy(srcg_hbm.at[pl.ds(base, CHUNK)], idx_v)
        pltpu.sync_copy(dst_hbm.at[pl.ds(base, CHUNK)], dst_v)
        pltpu.sync_copy(val_hbm.at[pl.ds(base, CHUNK)], val_v)
        pltpu.async_copy(mn_hbm.at[idx_v], rows_v, sem).wait()

        def scale_group(g, carry2):
            grp = val_v[pl.ds(g * L, L)]
            for lane in range(L):
                v = grp[lane]
                e = g * L + lane
                for j in range(D // L):
                    sl = pl.ds(j * L, L)
                    rows_v[e, sl] = rows_v[e, sl] * v
            return carry2

        lax.fori_loop(0, CHUNK // L, scale_group, 0)
        pltpu.sync_copy(rows_v, acc_sh.at[dst_v], add=True)
        return carry

    lax.fori_loop(0, NCHUNK, chunk_body, 0)
    plsc.subcore_barrier()

    pltpu.sync_copy(
        acc_sh.at[pl.ds(s * ROWS_PER_TILE, ROWS_PER_TILE)],
        out_hbm.at[pl.ds(c * NP + s * ROWS_PER_TILE, ROWS_PER_TILE)])


_sc_call = pl.kernel(
    _sc_body,
    out_type=jax.ShapeDtypeStruct((NC * NP, D), jnp.float32),
    mesh=plsc.VectorSubcoreMesh(core_axis_name="c", subcore_axis_name="s"),
    scratch_types=[
        pltpu.VMEM_SHARED((NP, D), jnp.float32),
        pltpu.VMEM((CHUNK,), jnp.int32),
        pltpu.VMEM((CHUNK,), jnp.int32),
        pltpu.VMEM((CHUNK,), jnp.float32),
        pltpu.VMEM((CHUNK, D), jnp.float32),
        pltpu.SemaphoreType.DMA,
    ],
)


def _finish_body(acc_ref, dense_ref, out_ref):
    out_ref[...] = jnp.maximum(acc_ref[0] + acc_ref[1] + dense_ref[...], 0.0)


def _finish_call(accs, dense):
    return pl.pallas_call(
        _finish_body,
        grid=(N // BN,),
        in_specs=[
            pl.BlockSpec((NC, BN, D), lambda i: (0, i, 0)),
            pl.BlockSpec((BN, D), lambda i: (i, 0)),
        ],
        out_specs=pl.BlockSpec((BN, D), lambda i: (i, 0)),
        out_shape=jax.ShapeDtypeStruct((N, D), jnp.float32),
    )(accs, dense)


def kernel(node_features, edge_index, adj_values, W_emb1, b_emb1, W_emb2,
           b_emb2, W_base, W_feat, W_res, alpha):
    mn, dense = _dense_call(node_features, W_emb1, b_emb1, W_emb2, b_emb2,
                            W_base, W_feat, W_res, alpha)
    mn_flat = mn.reshape(KHOP * N, D)

    src = edge_index[:, 0, :]
    dst = edge_index[:, 1, :]
    srcg = (src + (jnp.arange(KHOP, dtype=jnp.int32) * N)[:, None]).reshape(-1)
    dstf = dst.reshape(-1)
    valf = adj_values.reshape(-1)
    pad = ETP - ET
    srcg = jnp.concatenate([srcg, jnp.zeros((pad,), jnp.int32)])
    dstf = jnp.concatenate([dstf, jnp.zeros((pad,), jnp.int32)])
    valf = jnp.concatenate([valf, jnp.zeros((pad,), jnp.float32)])
    zeros = jnp.zeros((NP, D), jnp.float32)

    accs = _sc_call(mn_flat, srcg, dstf, valf, zeros)
    accs = accs.reshape(NC, NP, D)[:, :N, :]
    return _finish_call(accs, dense)
